# Initial kernel scaffold; baseline (speedup 1.0000x reference)
import functools

import jax
import jax.numpy as jnp
from jax import lax
from jax.experimental import pallas as pl
from jax.experimental.pallas import tpu as pltpu
from jax.experimental.pallas import tpu_sc as plsc

BATCH = 16384
N_FIELDS = 26
DIM = 128
FIELD_SIZE = 1000
TOTAL = BATCH * N_FIELDS

NUM_WORKERS = 32
PER_W = TOTAL // NUM_WORKERS
CHUNK = 128
NCHUNK = PER_W // CHUNK
LANES = 16

_mesh = plsc.VectorSubcoreMesh(core_axis_name="c", subcore_axis_name="s")


@functools.partial(
    pl.kernel,
    mesh=_mesh,
    out_type=jax.ShapeDtypeStruct((TOTAL, DIM), jnp.float32),
    scratch_types=[
        pltpu.VMEM((PER_W,), jnp.int32),
        pltpu.VMEM((PER_W,), jnp.int32),
        pltpu.VMEM((CHUNK, DIM), jnp.float32),
        pltpu.SemaphoreType.DMA,
    ],
)
def _emb_kernel(x_hbm, table_hbm, out_hbm, xv, idxv, rows, gsem):
    wid = lax.axis_index("s") * 2 + lax.axis_index("c")
    base = wid * PER_W

    pltpu.sync_copy(x_hbm.at[pl.ds(base, PER_W)], xv)

    lane = lax.iota(jnp.int32, (LANES,))

    def idx_body(j, _):
        pos = base + j * LANES
        field = (pos + lane) % N_FIELDS
        idxv[pl.ds(j * LANES, LANES)] = xv[pl.ds(j * LANES, LANES)] + field * FIELD_SIZE
        return 0

    lax.fori_loop(0, PER_W // LANES, idx_body, 0)

    def gather_body(c, _):
        pltpu.async_copy(
            table_hbm.at[idxv.at[pl.ds(c * CHUNK, CHUNK)]], rows, gsem
        ).wait()
        pltpu.sync_copy(rows, out_hbm.at[pl.ds(base + c * CHUNK, CHUNK), :])
        return 0

    lax.fori_loop(0, NCHUNK, gather_body, 0)


def kernel(x, table):
    out = _emb_kernel(x.reshape(-1).astype(jnp.int32), table)
    return out.reshape(BATCH, N_FIELDS, DIM)


# speedup vs baseline: 2.9546x; 2.9546x over previous
import functools

import jax
import jax.numpy as jnp
from jax import lax
from jax.experimental import pallas as pl
from jax.experimental.pallas import tpu as pltpu
from jax.experimental.pallas import tpu_sc as plsc

BATCH = 16384
N_FIELDS = 26
DIM = 128
FIELD_SIZE = 1000
TOTAL = BATCH * N_FIELDS

NUM_WORKERS = 32
PER_W = TOTAL // NUM_WORKERS
CHUNK = 128
NCHUNK = PER_W // CHUNK
LANES = 16

_mesh = plsc.VectorSubcoreMesh(core_axis_name="c", subcore_axis_name="s")


@functools.partial(
    pl.kernel,
    mesh=_mesh,
    out_type=jax.ShapeDtypeStruct((TOTAL, DIM), jnp.float32),
    scratch_types=[
        pltpu.VMEM((PER_W,), jnp.int32),
        pltpu.VMEM((PER_W,), jnp.int32),
        pltpu.VMEM((CHUNK, DIM), jnp.float32),
        pltpu.SemaphoreType.DMA,
    ],
)
def _emb_kernel(x_hbm, table_hbm, out_hbm, xv, idxv, rows, gsem):
    wid = lax.axis_index("s") * 2 + lax.axis_index("c")
    base = wid * PER_W

    pltpu.sync_copy(x_hbm.at[pl.ds(base, PER_W)], xv)

    lane = lax.iota(jnp.int32, LANES)

    def idx_body(j, _):
        pos = base + j * LANES
        field = (pos + lane) % N_FIELDS
        idxv[pl.ds(j * LANES, LANES)] = xv[pl.ds(j * LANES, LANES)] + field * FIELD_SIZE
        return 0

    lax.fori_loop(0, PER_W // LANES, idx_body, 0)

    def gather_body(c, _):
        pltpu.async_copy(
            table_hbm.at[idxv.at[pl.ds(c * CHUNK, CHUNK)]], rows, gsem
        ).wait()
        pltpu.sync_copy(rows, out_hbm.at[pl.ds(base + c * CHUNK, CHUNK), :])
        return 0

    lax.fori_loop(0, NCHUNK, gather_body, 0)


def kernel(x, table):
    out = _emb_kernel(x.reshape(-1).astype(jnp.int32), table)
    return out.reshape(BATCH, N_FIELDS, DIM)
